# initial kernel scaffold (unmeasured)
import jax
import jax.numpy as jnp
from jax import lax
from jax.experimental import pallas as pl
from jax.experimental.pallas import tpu as pltpu

N_DEV = 4
M = 4096
N = 8192
CHUNK = M // N_DEV
TILE = 256
NT = CHUNK // TILE


def _ar_body(partial_ref, alpha_ref, out_ref, comm_ref,
             rs_send, rs_recv, ag_send, ag_recv, local_sems,
             vmem_a, vmem_b):
    p = lax.axis_index("i")
    right = lax.rem(p + 1, N_DEV)
    left = lax.rem(p + N_DEV - 1, N_DEV)

    barrier = pltpu.get_barrier_semaphore()
    for nbr in (left, right):
        pl.semaphore_signal(barrier, inc=1, device_id=(nbr,),
                            device_id_type=pl.DeviceIdType.MESH)
    pl.semaphore_wait(barrier, 2)

    def load_pair(slot, row0):
        cp_a = pltpu.make_async_copy(
            comm_ref.at[slot, pl.ds(0, TILE), :].at[...] if False else
            comm_ref.at[slot].at[pl.ds(row0 % CHUNK, TILE), :],
            vmem_a, local_sems.at[0])
        return cp_a

    def add_chunk(slot, chunk_row0):
        for t in range(NT):
            cp_a = pltpu.make_async_copy(
                comm_ref.at[slot].at[pl.ds(t * TILE, TILE), :],
                vmem_a, local_sems.at[0])
            cp_b = pltpu.make_async_copy(
                partial_ref.at[pl.ds(chunk_row0 + t * TILE, TILE), :],
                vmem_b, local_sems.at[1])
            cp_a.start()
            cp_b.start()
            cp_a.wait()
            cp_b.wait()
            vmem_a[...] = vmem_a[...] + vmem_b[...]
            st = pltpu.make_async_copy(
                vmem_a, comm_ref.at[slot].at[pl.ds(t * TILE, TILE), :],
                local_sems.at[2])
            st.start()
            st.wait()

    def epilogue(slot, chunk_row0):
        alpha = alpha_ref[0, 0]
        for t in range(NT):
            cp_a = pltpu.make_async_copy(
                comm_ref.at[slot].at[pl.ds(t * TILE, TILE), :],
                vmem_a, local_sems.at[0])
            cp_b = pltpu.make_async_copy(
                partial_ref.at[pl.ds(chunk_row0 + t * TILE, TILE), :],
                vmem_b, local_sems.at[1])
            cp_a.start()
            cp_b.start()
            cp_a.wait()
            cp_b.wait()
            vmem_a[...] = jnp.maximum((vmem_a[...] + vmem_b[...]) * alpha, 0.0)
            st = pltpu.make_async_copy(
                vmem_a, out_ref.at[pl.ds(chunk_row0 + t * TILE, TILE), :],
                local_sems.at[2])
            st.start()
            st.wait()

    for h in range(N_DEV - 1):
        if h == 0:
            s = lax.rem(p + N_DEV - 1, N_DEV)
            src = partial_ref.at[pl.ds(s * CHUNK, CHUNK), :]
        else:
            src = comm_ref.at[h - 1]
        rdma = pltpu.make_async_remote_copy(
            src_ref=src,
            dst_ref=comm_ref.at[h],
            send_sem=rs_send.at[h],
            recv_sem=rs_recv.at[h],
            device_id=(right,),
            device_id_type=pl.DeviceIdType.MESH,
        )
        rdma.start()
        rdma.wait()
        r = lax.rem(p + 2 * N_DEV - 2 - h, N_DEV)
        if h < N_DEV - 2:
            add_chunk(h, r * CHUNK)
        else:
            epilogue(h, r * CHUNK)

    for g in range(N_DEV - 1):
        a = lax.rem(p + N_DEV - g, N_DEV)
        rdma = pltpu.make_async_remote_copy(
            src_ref=out_ref.at[pl.ds(a * CHUNK, CHUNK), :],
            dst_ref=out_ref.at[pl.ds(a * CHUNK, CHUNK), :],
            send_sem=ag_send.at[g],
            recv_sem=ag_recv.at[g],
            device_id=(right,),
            device_id_type=pl.DeviceIdType.MESH,
        )
        rdma.start()
        rdma.wait()


def kernel(x, w_mat, scale_x, scale_w):
    partial = lax.dot_general(
        x, w_mat,
        dimension_numbers=(((1,), (0,)), ((), ())),
        preferred_element_type=jnp.int32,
    ).astype(jnp.float32)
    alpha = (scale_x * scale_w).reshape(1, 1).astype(jnp.float32)

    return pl.pallas_call(
        _ar_body,
        out_shape=jax.ShapeDtypeStruct((M, N), jnp.float32),
        in_specs=[
            pl.BlockSpec(memory_space=pl.ANY),
            pl.BlockSpec(memory_space=pltpu.SMEM),
        ],
        out_specs=pl.BlockSpec(memory_space=pl.ANY),
        scratch_shapes=[
            pltpu.HBM((N_DEV - 1, CHUNK, N), jnp.float32),
            pltpu.SemaphoreType.DMA((N_DEV - 1,)),
            pltpu.SemaphoreType.DMA((N_DEV - 1,)),
            pltpu.SemaphoreType.DMA((N_DEV - 1,)),
            pltpu.SemaphoreType.DMA((N_DEV - 1,)),
            pltpu.SemaphoreType.DMA((3,)),
            pltpu.VMEM((TILE, N), jnp.float32),
            pltpu.VMEM((TILE, N), jnp.float32),
        ],
        compiler_params=pltpu.CompilerParams(collective_id=0),
    )(partial, alpha)


# baseline (device time: 2469446 ns/iter reference)
import jax
import jax.numpy as jnp
from jax import lax
from jax.experimental import pallas as pl
from jax.experimental.pallas import tpu as pltpu

N_DEV = 4
M = 4096
N = 8192
CHUNK = M // N_DEV
TILE = 256
NT = CHUNK // TILE


def _ar_body(partial_ref, alpha_ref, out_ref, comm_ref,
             rs_send, rs_recv, ag_send, ag_recv, local_sems,
             vmem_a, vmem_b):
    p = lax.axis_index("i")
    right = lax.rem(p + 1, N_DEV)
    left = lax.rem(p + N_DEV - 1, N_DEV)

    barrier = pltpu.get_barrier_semaphore()
    for nbr in (left, right):
        pl.semaphore_signal(barrier, inc=1, device_id=(nbr,),
                            device_id_type=pl.DeviceIdType.MESH)
    pl.semaphore_wait(barrier, 2)

    def add_chunk(slot, chunk_row0):
        for t in range(NT):
            cp_a = pltpu.make_async_copy(
                comm_ref.at[slot].at[pl.ds(t * TILE, TILE), :],
                vmem_a, local_sems.at[0])
            cp_b = pltpu.make_async_copy(
                partial_ref.at[pl.ds(chunk_row0 + t * TILE, TILE), :],
                vmem_b, local_sems.at[1])
            cp_a.start()
            cp_b.start()
            cp_a.wait()
            cp_b.wait()
            vmem_a[...] = vmem_a[...] + vmem_b[...]
            st = pltpu.make_async_copy(
                vmem_a, comm_ref.at[slot].at[pl.ds(t * TILE, TILE), :],
                local_sems.at[2])
            st.start()
            st.wait()

    def epilogue(slot, chunk_row0):
        alpha = alpha_ref[0, 0]
        for t in range(NT):
            cp_a = pltpu.make_async_copy(
                comm_ref.at[slot].at[pl.ds(t * TILE, TILE), :],
                vmem_a, local_sems.at[0])
            cp_b = pltpu.make_async_copy(
                partial_ref.at[pl.ds(chunk_row0 + t * TILE, TILE), :],
                vmem_b, local_sems.at[1])
            cp_a.start()
            cp_b.start()
            cp_a.wait()
            cp_b.wait()
            vmem_a[...] = jnp.maximum((vmem_a[...] + vmem_b[...]) * alpha, 0.0)
            st = pltpu.make_async_copy(
                vmem_a, out_ref.at[pl.ds(chunk_row0 + t * TILE, TILE), :],
                local_sems.at[2])
            st.start()
            st.wait()

    for h in range(N_DEV - 1):
        if h == 0:
            s = lax.rem(p + N_DEV - 1, N_DEV)
            src = partial_ref.at[pl.ds(s * CHUNK, CHUNK), :]
        else:
            src = comm_ref.at[h - 1]
        rdma = pltpu.make_async_remote_copy(
            src_ref=src,
            dst_ref=comm_ref.at[h],
            send_sem=rs_send.at[h],
            recv_sem=rs_recv.at[h],
            device_id=(right,),
            device_id_type=pl.DeviceIdType.MESH,
        )
        rdma.start()
        rdma.wait()
        r = lax.rem(p + 2 * N_DEV - 2 - h, N_DEV)
        if h < N_DEV - 2:
            add_chunk(h, r * CHUNK)
        else:
            epilogue(h, r * CHUNK)

    for g in range(N_DEV - 1):
        a = lax.rem(p + N_DEV - g, N_DEV)
        rdma = pltpu.make_async_remote_copy(
            src_ref=out_ref.at[pl.ds(a * CHUNK, CHUNK), :],
            dst_ref=out_ref.at[pl.ds(a * CHUNK, CHUNK), :],
            send_sem=ag_send.at[g],
            recv_sem=ag_recv.at[g],
            device_id=(right,),
            device_id_type=pl.DeviceIdType.MESH,
        )
        rdma.start()
        rdma.wait()


def kernel(x, w_mat, scale_x, scale_w):
    partial = lax.dot_general(
        x, w_mat,
        dimension_numbers=(((1,), (0,)), ((), ())),
        preferred_element_type=jnp.int32,
    ).astype(jnp.float32)
    alpha = (scale_x * scale_w).reshape(1, 1).astype(jnp.float32)

    out, _ = pl.pallas_call(
        _ar_body,
        out_shape=[
            jax.ShapeDtypeStruct((M, N), jnp.float32),
            jax.ShapeDtypeStruct((N_DEV - 1, CHUNK, N), jnp.float32),
        ],
        in_specs=[
            pl.BlockSpec(memory_space=pl.ANY),
            pl.BlockSpec(memory_space=pltpu.SMEM),
        ],
        out_specs=[
            pl.BlockSpec(memory_space=pl.ANY),
            pl.BlockSpec(memory_space=pl.ANY),
        ],
        scratch_shapes=[
            pltpu.SemaphoreType.DMA((N_DEV - 1,)),
            pltpu.SemaphoreType.DMA((N_DEV - 1,)),
            pltpu.SemaphoreType.DMA((N_DEV - 1,)),
            pltpu.SemaphoreType.DMA((N_DEV - 1,)),
            pltpu.SemaphoreType.DMA((3,)),
            pltpu.VMEM((TILE, N), jnp.float32),
            pltpu.VMEM((TILE, N), jnp.float32),
        ],
        compiler_params=pltpu.CompilerParams(collective_id=0),
    )(partial, alpha)
    return out


# device time: 1411866 ns/iter; 1.7491x vs baseline; 1.7491x over previous
import jax
import jax.numpy as jnp
from jax import lax
from jax.experimental import pallas as pl
from jax.experimental.pallas import tpu as pltpu

N_DEV = 4
M = 4096
N = 8192
N2 = N // 2
CHUNK = M // N_DEV
TILE = 256
NT = CHUNK // TILE


def _ar_body(partial_ref, alpha_ref, out_ref, comm_ref,
             rs_send_r, rs_recv_r, ag_send_r, ag_recv_r,
             rs_send_l, rs_recv_l, ag_send_l, ag_recv_l,
             local_sems, vmem_a, vmem_b):
    p = lax.axis_index("i")
    right = lax.rem(p + 1, N_DEV)
    left = lax.rem(p + N_DEV - 1, N_DEV)

    barrier = pltpu.get_barrier_semaphore()
    for nbr in (left, right):
        pl.semaphore_signal(barrier, inc=1, device_id=(nbr,),
                            device_id_type=pl.DeviceIdType.MESH)
    pl.semaphore_wait(barrier, 2)

    def add_chunk(ring, slot, chunk_row0, col0):
        for t in range(NT):
            cp_a = pltpu.make_async_copy(
                comm_ref.at[ring, slot].at[pl.ds(t * TILE, TILE), :],
                vmem_a, local_sems.at[0])
            cp_b = pltpu.make_async_copy(
                partial_ref.at[pl.ds(chunk_row0 + t * TILE, TILE),
                               pl.ds(col0, N2)],
                vmem_b, local_sems.at[1])
            cp_a.start()
            cp_b.start()
            cp_a.wait()
            cp_b.wait()
            vmem_a[...] = vmem_a[...] + vmem_b[...]
            st = pltpu.make_async_copy(
                vmem_a, comm_ref.at[ring, slot].at[pl.ds(t * TILE, TILE), :],
                local_sems.at[2])
            st.start()
            st.wait()

    def epilogue(ring, slot, chunk_row0, col0):
        alpha = alpha_ref[0, 0]
        for t in range(NT):
            cp_a = pltpu.make_async_copy(
                comm_ref.at[ring, slot].at[pl.ds(t * TILE, TILE), :],
                vmem_a, local_sems.at[0])
            cp_b = pltpu.make_async_copy(
                partial_ref.at[pl.ds(chunk_row0 + t * TILE, TILE),
                               pl.ds(col0, N2)],
                vmem_b, local_sems.at[1])
            cp_a.start()
            cp_b.start()
            cp_a.wait()
            cp_b.wait()
            vmem_a[...] = jnp.maximum((vmem_a[...] + vmem_b[...]) * alpha, 0.0)
            st = pltpu.make_async_copy(
                vmem_a, out_ref.at[pl.ds(chunk_row0 + t * TILE, TILE),
                                   pl.ds(col0, N2)],
                local_sems.at[2])
            st.start()
            st.wait()

    for h in range(N_DEV - 1):
        if h == 0:
            sr = lax.rem(p + N_DEV - 1, N_DEV)
            sl = lax.rem(p + 1, N_DEV)
            src_r = partial_ref.at[pl.ds(sr * CHUNK, CHUNK), pl.ds(0, N2)]
            src_l = partial_ref.at[pl.ds(sl * CHUNK, CHUNK), pl.ds(N2, N2)]
        else:
            src_r = comm_ref.at[0, h - 1]
            src_l = comm_ref.at[1, h - 1]
        rdma_r = pltpu.make_async_remote_copy(
            src_ref=src_r, dst_ref=comm_ref.at[0, h],
            send_sem=rs_send_r.at[h], recv_sem=rs_recv_r.at[h],
            device_id=(right,), device_id_type=pl.DeviceIdType.MESH,
        )
        rdma_l = pltpu.make_async_remote_copy(
            src_ref=src_l, dst_ref=comm_ref.at[1, h],
            send_sem=rs_send_l.at[h], recv_sem=rs_recv_l.at[h],
            device_id=(left,), device_id_type=pl.DeviceIdType.MESH,
        )
        rdma_r.start()
        rdma_l.start()
        rdma_r.wait()
        rdma_l.wait()
        rr = lax.rem(p + 2 * N_DEV - 2 - h, N_DEV)
        rl = lax.rem(p + 2 + h, N_DEV)
        if h < N_DEV - 2:
            add_chunk(0, h, rr * CHUNK, 0)
            add_chunk(1, h, rl * CHUNK, N2)
        else:
            epilogue(0, h, rr * CHUNK, 0)
            epilogue(1, h, rl * CHUNK, N2)

    for g in range(N_DEV - 1):
        ar = lax.rem(p + N_DEV - g, N_DEV)
        al = lax.rem(p + g, N_DEV)
        rdma_r = pltpu.make_async_remote_copy(
            src_ref=out_ref.at[pl.ds(ar * CHUNK, CHUNK), pl.ds(0, N2)],
            dst_ref=out_ref.at[pl.ds(ar * CHUNK, CHUNK), pl.ds(0, N2)],
            send_sem=ag_send_r.at[g], recv_sem=ag_recv_r.at[g],
            device_id=(right,), device_id_type=pl.DeviceIdType.MESH,
        )
        rdma_l = pltpu.make_async_remote_copy(
            src_ref=out_ref.at[pl.ds(al * CHUNK, CHUNK), pl.ds(N2, N2)],
            dst_ref=out_ref.at[pl.ds(al * CHUNK, CHUNK), pl.ds(N2, N2)],
            send_sem=ag_send_l.at[g], recv_sem=ag_recv_l.at[g],
            device_id=(left,), device_id_type=pl.DeviceIdType.MESH,
        )
        rdma_r.start()
        rdma_l.start()
        rdma_r.wait()
        rdma_l.wait()


def kernel(x, w_mat, scale_x, scale_w):
    partial = lax.dot_general(
        x, w_mat,
        dimension_numbers=(((1,), (0,)), ((), ())),
        preferred_element_type=jnp.int32,
    ).astype(jnp.float32)
    alpha = (scale_x * scale_w).reshape(1, 1).astype(jnp.float32)

    out, _ = pl.pallas_call(
        _ar_body,
        out_shape=[
            jax.ShapeDtypeStruct((M, N), jnp.float32),
            jax.ShapeDtypeStruct((2, N_DEV - 1, CHUNK, N2), jnp.float32),
        ],
        in_specs=[
            pl.BlockSpec(memory_space=pl.ANY),
            pl.BlockSpec(memory_space=pltpu.SMEM),
        ],
        out_specs=[
            pl.BlockSpec(memory_space=pl.ANY),
            pl.BlockSpec(memory_space=pl.ANY),
        ],
        scratch_shapes=[
            pltpu.SemaphoreType.DMA((N_DEV - 1,)),
            pltpu.SemaphoreType.DMA((N_DEV - 1,)),
            pltpu.SemaphoreType.DMA((N_DEV - 1,)),
            pltpu.SemaphoreType.DMA((N_DEV - 1,)),
            pltpu.SemaphoreType.DMA((N_DEV - 1,)),
            pltpu.SemaphoreType.DMA((N_DEV - 1,)),
            pltpu.SemaphoreType.DMA((N_DEV - 1,)),
            pltpu.SemaphoreType.DMA((N_DEV - 1,)),
            pltpu.SemaphoreType.DMA((3,)),
            pltpu.VMEM((TILE, N2), jnp.float32),
            pltpu.VMEM((TILE, N2), jnp.float32),
        ],
        compiler_params=pltpu.CompilerParams(collective_id=0),
    )(partial, alpha)
    return out


# device time: 1264098 ns/iter; 1.9535x vs baseline; 1.1169x over previous
import jax
import jax.numpy as jnp
from jax import lax
from jax.experimental import pallas as pl
from jax.experimental.pallas import tpu as pltpu

N_DEV = 4
M = 4096
N = 8192
N2 = N // 2
CHUNK = M // N_DEV
NTT = 4
TILE = CHUNK // NTT


def _ar_body(partial_ref, alpha_ref, out_ref, comm_ref,
             rs_send_sems, rs_recv_sems, ag_send_sems, ag_recv_sems,
             local_sems, va0, vb0, va1, vb1):
    p = lax.axis_index("i")
    right = lax.rem(p + 1, N_DEV)
    left = lax.rem(p + N_DEV - 1, N_DEV)
    alpha = alpha_ref[0, 0]
    vmem_a = (va0, va1)
    vmem_b = (vb0, vb1)

    barrier = pltpu.get_barrier_semaphore()
    for nbr in (left, right):
        pl.semaphore_signal(barrier, inc=1, device_id=(nbr,),
                            device_id_type=pl.DeviceIdType.MESH)
    pl.semaphore_wait(barrier, 2)

    started = []

    def comm_tile(ring, h, t):
        return comm_ref.at[ring, h].at[pl.ds(t * TILE, TILE), :]

    def out_tile(ring, chunk, t):
        return out_ref.at[pl.ds(chunk * CHUNK + t * TILE, TILE),
                          pl.ds((0, N2)[ring], N2)]

    def rs_send(ring, h, t, src):
        rdma = pltpu.make_async_remote_copy(
            src_ref=src, dst_ref=comm_tile(ring, h, t),
            send_sem=rs_send_sems.at[ring, h, t],
            recv_sem=rs_recv_sems.at[ring, h, t],
            device_id=((right, left)[ring],),
            device_id_type=pl.DeviceIdType.MESH,
        )
        rdma.start()
        started.append(rdma)

    def rs_recv_wait(ring, h, t):
        pltpu.make_async_remote_copy(
            src_ref=comm_tile(ring, h, t), dst_ref=comm_tile(ring, h, t),
            send_sem=rs_send_sems.at[ring, h, t],
            recv_sem=rs_recv_sems.at[ring, h, t],
            device_id=((left, right)[ring],),
            device_id_type=pl.DeviceIdType.MESH,
        ).wait_recv()

    def ag_send(ring, g, t, chunk):
        src = out_tile(ring, chunk, t)
        rdma = pltpu.make_async_remote_copy(
            src_ref=src, dst_ref=src,
            send_sem=ag_send_sems.at[ring, g, t],
            recv_sem=ag_recv_sems.at[ring, g, t],
            device_id=((right, left)[ring],),
            device_id_type=pl.DeviceIdType.MESH,
        )
        rdma.start()
        started.append(rdma)

    def ag_recv_wait(ring, g, t, chunk):
        dst = out_tile(ring, chunk, t)
        pltpu.make_async_remote_copy(
            src_ref=dst, dst_ref=dst,
            send_sem=ag_send_sems.at[ring, g, t],
            recv_sem=ag_recv_sems.at[ring, g, t],
            device_id=((left, right)[ring],),
            device_id_type=pl.DeviceIdType.MESH,
        ).wait_recv()

    sr = lax.rem(p + N_DEV - 1, N_DEV)
    sl = lax.rem(p + 1, N_DEV)
    for t in range(NTT):
        rs_send(0, 0, t, partial_ref.at[pl.ds(sr * CHUNK + t * TILE, TILE),
                                        pl.ds(0, N2)])
        rs_send(1, 0, t, partial_ref.at[pl.ds(sl * CHUNK + t * TILE, TILE),
                                        pl.ds(N2, N2)])

    for h in range(N_DEV - 1):
        rcv_chunk = (lax.rem(p + 2 * N_DEV - 2 - h, N_DEV),
                     lax.rem(p + 2 + h, N_DEV))
        last = h == N_DEV - 2
        for t in range(NTT):
            loads = [None, None]
            for ring in (0, 1):
                rs_recv_wait(ring, h, t)
                row0 = rcv_chunk[ring] * CHUNK + t * TILE
                cp_a = pltpu.make_async_copy(
                    comm_tile(ring, h, t), vmem_a[ring],
                    local_sems.at[2 * ring])
                cp_b = pltpu.make_async_copy(
                    partial_ref.at[pl.ds(row0, TILE),
                                   pl.ds((0, N2)[ring], N2)],
                    vmem_b[ring], local_sems.at[2 * ring + 1])
                cp_a.start()
                cp_b.start()
                loads[ring] = (cp_a, cp_b)
            stores = [None, None]
            for ring in (0, 1):
                cp_a, cp_b = loads[ring]
                cp_a.wait()
                cp_b.wait()
                if not last:
                    vmem_a[ring][...] = vmem_a[ring][...] + vmem_b[ring][...]
                    dst = comm_tile(ring, h, t)
                else:
                    vmem_a[ring][...] = jnp.maximum(
                        (vmem_a[ring][...] + vmem_b[ring][...]) * alpha, 0.0)
                    dst = out_tile(ring, rcv_chunk[ring], t)
                st = pltpu.make_async_copy(vmem_a[ring], dst,
                                           local_sems.at[4 + ring])
                st.start()
                stores[ring] = (st, dst)
            for ring in (0, 1):
                st, dst = stores[ring]
                st.wait()
                if not last:
                    rs_send(ring, h + 1, t, dst)
                else:
                    ag_send(ring, 0, t, rcv_chunk[ring])

    for g in range(N_DEV - 1):
        rcv_chunk = (lax.rem(p + 2 * N_DEV - 1 - g, N_DEV),
                     lax.rem(p + 1 + g, N_DEV))
        for t in range(NTT):
            for ring in (0, 1):
                ag_recv_wait(ring, g, t, rcv_chunk[ring])
                if g < N_DEV - 2:
                    ag_send(ring, g + 1, t, rcv_chunk[ring])

    for rdma in started:
        rdma.wait_send()


def kernel(x, w_mat, scale_x, scale_w):
    partial = lax.dot_general(
        x, w_mat,
        dimension_numbers=(((1,), (0,)), ((), ())),
        preferred_element_type=jnp.int32,
    ).astype(jnp.float32)
    alpha = (scale_x * scale_w).reshape(1, 1).astype(jnp.float32)

    out, _ = pl.pallas_call(
        _ar_body,
        out_shape=[
            jax.ShapeDtypeStruct((M, N), jnp.float32),
            jax.ShapeDtypeStruct((2, N_DEV - 1, CHUNK, N2), jnp.float32),
        ],
        in_specs=[
            pl.BlockSpec(memory_space=pl.ANY),
            pl.BlockSpec(memory_space=pltpu.SMEM),
        ],
        out_specs=[
            pl.BlockSpec(memory_space=pl.ANY),
            pl.BlockSpec(memory_space=pl.ANY),
        ],
        scratch_shapes=[
            pltpu.SemaphoreType.DMA((2, N_DEV - 1, NTT)),
            pltpu.SemaphoreType.DMA((2, N_DEV - 1, NTT)),
            pltpu.SemaphoreType.DMA((2, N_DEV - 1, NTT)),
            pltpu.SemaphoreType.DMA((2, N_DEV - 1, NTT)),
            pltpu.SemaphoreType.DMA((6,)),
            pltpu.VMEM((TILE, N2), jnp.float32),
            pltpu.VMEM((TILE, N2), jnp.float32),
            pltpu.VMEM((TILE, N2), jnp.float32),
            pltpu.VMEM((TILE, N2), jnp.float32),
        ],
        compiler_params=pltpu.CompilerParams(collective_id=0),
    )(partial, alpha)
    return out


# device time: 729870 ns/iter; 3.3834x vs baseline; 1.7319x over previous
import jax
import jax.numpy as jnp
from jax import lax
from jax.experimental import pallas as pl
from jax.experimental.pallas import tpu as pltpu

N_DEV = 4
M = 4096
N = 8192
N2 = N // 2
CHUNK = M // N_DEV
NTT = 4
TILE = CHUNK // NTT
CDT = jnp.bfloat16


def _ar_body(partial_ref, out_ref, rs_comm, ag_comm, ag_src,
             rs_send_sems, rs_recv_sems, ag_send_sems, ag_recv_sems,
             local_sems, a16_0, b16_0, s16_0, f32_0,
             a16_1, b16_1, s16_1, f32_1):
    p = lax.axis_index("i")
    right = lax.rem(p + 1, N_DEV)
    left = lax.rem(p + N_DEV - 1, N_DEV)
    va16 = (a16_0, a16_1)
    vb16 = (b16_0, b16_1)
    vs16 = (s16_0, s16_1)
    vf32 = (f32_0, f32_1)

    barrier = pltpu.get_barrier_semaphore()
    for nbr in (left, right):
        pl.semaphore_signal(barrier, inc=1, device_id=(nbr,),
                            device_id_type=pl.DeviceIdType.MESH)
    pl.semaphore_wait(barrier, 2)

    started = []

    def rs_tile(ring, h, t):
        return rs_comm.at[ring, h].at[pl.ds(t * TILE, TILE), :]

    def ag_tile(ring, g, t):
        return ag_comm.at[ring, g].at[pl.ds(t * TILE, TILE), :]

    def out_tile(ring, chunk, t):
        return out_ref.at[pl.ds(chunk * CHUNK + t * TILE, TILE),
                          pl.ds((0, N2)[ring], N2)]

    def rs_send(ring, h, t, src):
        rdma = pltpu.make_async_remote_copy(
            src_ref=src, dst_ref=rs_tile(ring, h, t),
            send_sem=rs_send_sems.at[ring, h, t],
            recv_sem=rs_recv_sems.at[ring, h, t],
            device_id=((right, left)[ring],),
            device_id_type=pl.DeviceIdType.MESH,
        )
        rdma.start()
        started.append(rdma)

    def rs_recv_wait(ring, h, t):
        pltpu.make_async_remote_copy(
            src_ref=rs_tile(ring, h, t), dst_ref=rs_tile(ring, h, t),
            send_sem=rs_send_sems.at[ring, h, t],
            recv_sem=rs_recv_sems.at[ring, h, t],
            device_id=((left, right)[ring],),
            device_id_type=pl.DeviceIdType.MESH,
        ).wait_recv()

    def ag_send(ring, g, t, src):
        rdma = pltpu.make_async_remote_copy(
            src_ref=src, dst_ref=ag_tile(ring, g, t),
            send_sem=ag_send_sems.at[ring, g, t],
            recv_sem=ag_recv_sems.at[ring, g, t],
            device_id=((right, left)[ring],),
            device_id_type=pl.DeviceIdType.MESH,
        )
        rdma.start()
        started.append(rdma)

    def ag_recv_wait(ring, g, t):
        pltpu.make_async_remote_copy(
            src_ref=ag_tile(ring, g, t), dst_ref=ag_tile(ring, g, t),
            send_sem=ag_send_sems.at[ring, g, t],
            recv_sem=ag_recv_sems.at[ring, g, t],
            device_id=((left, right)[ring],),
            device_id_type=pl.DeviceIdType.MESH,
        ).wait_recv()

    sr = lax.rem(p + N_DEV - 1, N_DEV)
    sl = lax.rem(p + 1, N_DEV)
    for t in range(NTT):
        rs_send(0, 0, t, partial_ref.at[pl.ds(sr * CHUNK + t * TILE, TILE),
                                        pl.ds(0, N2)])
        rs_send(1, 0, t, partial_ref.at[pl.ds(sl * CHUNK + t * TILE, TILE),
                                        pl.ds(N2, N2)])

    for h in range(N_DEV - 1):
        rcv_chunk = (lax.rem(p + 2 * N_DEV - 2 - h, N_DEV),
                     lax.rem(p + 2 + h, N_DEV))
        last = h == N_DEV - 2
        for t in range(NTT):
            loads = [None, None]
            for ring in (0, 1):
                rs_recv_wait(ring, h, t)
                row0 = rcv_chunk[ring] * CHUNK + t * TILE
                cp_a = pltpu.make_async_copy(
                    rs_tile(ring, h, t), va16[ring], local_sems.at[2 * ring])
                cp_b = pltpu.make_async_copy(
                    partial_ref.at[pl.ds(row0, TILE),
                                   pl.ds((0, N2)[ring], N2)],
                    vb16[ring], local_sems.at[2 * ring + 1])
                cp_a.start()
                cp_b.start()
                loads[ring] = (cp_a, cp_b)
            stores = [None, None]
            for ring in (0, 1):
                cp_a, cp_b = loads[ring]
                cp_a.wait()
                cp_b.wait()
                acc = (va16[ring][...].astype(jnp.float32)
                       + vb16[ring][...].astype(jnp.float32))
                if not last:
                    vs16[ring][...] = acc.astype(CDT)
                    dst = rs_tile(ring, h, t)
                else:
                    y = jnp.maximum(acc, 0.0)
                    vs16[ring][...] = y.astype(CDT)
                    vf32[ring][...] = y * 4.0
                    dst = ag_src.at[ring].at[pl.ds(t * TILE, TILE), :]
                st = pltpu.make_async_copy(vs16[ring], dst,
                                           local_sems.at[4 + ring])
                st.start()
                stores[ring] = (st, dst)
            for ring in (0, 1):
                st, dst = stores[ring]
                st.wait()
                if not last:
                    rs_send(ring, h + 1, t, dst)
                else:
                    ag_send(ring, 0, t, dst)
                    st32 = pltpu.make_async_copy(
                        vf32[ring], out_tile(ring, rcv_chunk[ring], t),
                        local_sems.at[6 + ring])
                    st32.start()
                    st32.wait()

    for g in range(N_DEV - 1):
        rcv_chunk = (lax.rem(p + 2 * N_DEV - 1 - g, N_DEV),
                     lax.rem(p + 1 + g, N_DEV))
        for t in range(NTT):
            loads = [None, None]
            for ring in (0, 1):
                ag_recv_wait(ring, g, t)
                if g < N_DEV - 2:
                    ag_send(ring, g + 1, t, ag_tile(ring, g, t))
                cp = pltpu.make_async_copy(
                    ag_tile(ring, g, t), va16[ring], local_sems.at[2 * ring])
                cp.start()
                loads[ring] = cp
            stores = [None, None]
            for ring in (0, 1):
                loads[ring].wait()
                vf32[ring][...] = va16[ring][...].astype(jnp.float32) * 4.0
                st = pltpu.make_async_copy(
                    vf32[ring], out_tile(ring, rcv_chunk[ring], t),
                    local_sems.at[6 + ring])
                st.start()
                stores[ring] = st
            for ring in (0, 1):
                stores[ring].wait()

    for rdma in started:
        rdma.wait_send()


def kernel(x, w_mat, scale_x, scale_w):
    acc = lax.dot_general(
        x, w_mat,
        dimension_numbers=(((1,), (0,)), ((), ())),
        preferred_element_type=jnp.int32,
    ).astype(jnp.float32)
    alpha = (scale_x[0] * scale_w[0]).astype(jnp.float32)
    partial = (acc * (alpha * 0.25)).astype(CDT)

    out = pl.pallas_call(
        _ar_body,
        out_shape=[
            jax.ShapeDtypeStruct((M, N), jnp.float32),
            jax.ShapeDtypeStruct((2, N_DEV - 1, CHUNK, N2), CDT),
            jax.ShapeDtypeStruct((2, N_DEV - 1, CHUNK, N2), CDT),
            jax.ShapeDtypeStruct((2, CHUNK, N2), CDT),
        ],
        in_specs=[pl.BlockSpec(memory_space=pl.ANY)],
        out_specs=[pl.BlockSpec(memory_space=pl.ANY)] * 4,
        scratch_shapes=[
            pltpu.SemaphoreType.DMA((2, N_DEV - 1, NTT)),
            pltpu.SemaphoreType.DMA((2, N_DEV - 1, NTT)),
            pltpu.SemaphoreType.DMA((2, N_DEV - 1, NTT)),
            pltpu.SemaphoreType.DMA((2, N_DEV - 1, NTT)),
            pltpu.SemaphoreType.DMA((8,)),
            pltpu.VMEM((TILE, N2), CDT),
            pltpu.VMEM((TILE, N2), CDT),
            pltpu.VMEM((TILE, N2), CDT),
            pltpu.VMEM((TILE, N2), jnp.float32),
            pltpu.VMEM((TILE, N2), CDT),
            pltpu.VMEM((TILE, N2), CDT),
            pltpu.VMEM((TILE, N2), CDT),
            pltpu.VMEM((TILE, N2), jnp.float32),
        ],
        compiler_params=pltpu.CompilerParams(collective_id=0),
    )(partial)[0]
    return out


# device time: 648760 ns/iter; 3.8064x vs baseline; 1.1250x over previous
import jax
import jax.numpy as jnp
from jax import lax
from jax.experimental import pallas as pl
from jax.experimental.pallas import tpu as pltpu

N_DEV = 4
M = 4096
K = 1024
N = 8192
N2 = N // 2
CHUNK = M // N_DEV
NTT = 4
TILE = CHUNK // NTT
CDT = jnp.bfloat16


def _ar_body(x_ref, w_ref, alpha_ref, out_ref,
             rs_comm, ag_comm, ag_src, stage,
             rs_send_sems, rs_recv_sems, ag_send_sems, ag_recv_sems,
             local_sems, a16_0, a16_1, b16_0, b16_1, s16_0, s16_1,
             f32_0, f32_1, xi8_0, xi8_1):
    p = lax.axis_index("i")
    right = lax.rem(p + 1, N_DEV)
    left = lax.rem(p + N_DEV - 1, N_DEV)
    alpha4 = alpha_ref[0, 0]
    va16 = (a16_0, a16_1)
    vb16 = (b16_0, b16_1)
    vs16 = (s16_0, s16_1)
    vf32 = (f32_0, f32_1)
    xi8 = (xi8_0, xi8_1)

    barrier = pltpu.get_barrier_semaphore()
    for nbr in (left, right):
        pl.semaphore_signal(barrier, inc=1, device_id=(nbr,),
                            device_id_type=pl.DeviceIdType.MESH)
    pl.semaphore_wait(barrier, 2)

    started = []

    def rs_tile(ring, h, t):
        return rs_comm.at[ring, h].at[pl.ds(t * TILE, TILE), :]

    def ag_tile(ring, g, t):
        return ag_comm.at[ring, g].at[pl.ds(t * TILE, TILE), :]

    def out_tile(ring, chunk, t):
        return out_ref.at[pl.ds(chunk * CHUNK + t * TILE, TILE),
                          pl.ds((0, N2)[ring], N2)]

    def rs_send(ring, h, t, src):
        rdma = pltpu.make_async_remote_copy(
            src_ref=src, dst_ref=rs_tile(ring, h, t),
            send_sem=rs_send_sems.at[ring, h, t],
            recv_sem=rs_recv_sems.at[ring, h, t],
            device_id=((right, left)[ring],),
            device_id_type=pl.DeviceIdType.MESH,
        )
        rdma.start()
        started.append(rdma)

    def rs_recv_wait(ring, h, t):
        pltpu.make_async_remote_copy(
            src_ref=rs_tile(ring, h, t), dst_ref=rs_tile(ring, h, t),
            send_sem=rs_send_sems.at[ring, h, t],
            recv_sem=rs_recv_sems.at[ring, h, t],
            device_id=((left, right)[ring],),
            device_id_type=pl.DeviceIdType.MESH,
        ).wait_recv()

    def ag_send(ring, g, t, src):
        rdma = pltpu.make_async_remote_copy(
            src_ref=src, dst_ref=ag_tile(ring, g, t),
            send_sem=ag_send_sems.at[ring, g, t],
            recv_sem=ag_recv_sems.at[ring, g, t],
            device_id=((right, left)[ring],),
            device_id_type=pl.DeviceIdType.MESH,
        )
        rdma.start()
        started.append(rdma)

    def ag_recv_wait(ring, g, t):
        pltpu.make_async_remote_copy(
            src_ref=ag_tile(ring, g, t), dst_ref=ag_tile(ring, g, t),
            send_sem=ag_send_sems.at[ring, g, t],
            recv_sem=ag_recv_sems.at[ring, g, t],
            device_id=((left, right)[ring],),
            device_id_type=pl.DeviceIdType.MESH,
        ).wait_recv()

    def compute_tile(ring, chunk, t):
        row0 = chunk * CHUNK + t * TILE
        cp = pltpu.make_async_copy(
            x_ref.at[pl.ds(row0, TILE), :], xi8[ring],
            local_sems.at[6 + ring])
        cp.start()
        cp.wait()
        acc = lax.dot_general(
            xi8[ring][...], w_ref[:, (0, N2)[ring]:(N2, N)[ring]],
            dimension_numbers=(((1,), (0,)), ((), ())),
            preferred_element_type=jnp.int32,
        )
        vb16[ring][...] = (acc.astype(jnp.float32) * alpha4).astype(CDT)

    send_chunk = (lax.rem(p + N_DEV - 1, N_DEV), lax.rem(p + 1, N_DEV))
    for t in range(NTT):
        for ring in (0, 1):
            compute_tile(ring, send_chunk[ring], t)
            dst = stage.at[ring].at[pl.ds(t * TILE, TILE), :]
            st = pltpu.make_async_copy(vb16[ring], dst,
                                       local_sems.at[4 + ring])
            st.start()
            st.wait()
            rs_send(ring, 0, t, dst)

    for h in range(N_DEV - 1):
        rcv_chunk = (lax.rem(p + 2 * N_DEV - 2 - h, N_DEV),
                     lax.rem(p + 2 + h, N_DEV))
        last = h == N_DEV - 2
        for t in range(NTT):
            for ring in (0, 1):
                compute_tile(ring, rcv_chunk[ring], t)
            loads = [None, None]
            for ring in (0, 1):
                rs_recv_wait(ring, h, t)
                cp_a = pltpu.make_async_copy(
                    rs_tile(ring, h, t), va16[ring], local_sems.at[2 * ring])
                cp_a.start()
                loads[ring] = cp_a
            stores = [None, None]
            for ring in (0, 1):
                loads[ring].wait()
                acc = (va16[ring][...].astype(jnp.float32)
                       + vb16[ring][...].astype(jnp.float32))
                if not last:
                    vs16[ring][...] = acc.astype(CDT)
                    dst = rs_tile(ring, h, t)
                else:
                    y = jnp.maximum(acc, 0.0)
                    vs16[ring][...] = y.astype(CDT)
                    vf32[ring][...] = y * 4.0
                    dst = ag_src.at[ring].at[pl.ds(t * TILE, TILE), :]
                st = pltpu.make_async_copy(vs16[ring], dst,
                                           local_sems.at[4 + ring])
                st.start()
                stores[ring] = (st, dst)
            for ring in (0, 1):
                st, dst = stores[ring]
                st.wait()
                if not last:
                    rs_send(ring, h + 1, t, dst)
                else:
                    ag_send(ring, 0, t, dst)
                    st32 = pltpu.make_async_copy(
                        vf32[ring], out_tile(ring, rcv_chunk[ring], t),
                        local_sems.at[8 + ring])
                    st32.start()
                    st32.wait()

    for g in range(N_DEV - 1):
        rcv_chunk = (lax.rem(p + 2 * N_DEV - 1 - g, N_DEV),
                     lax.rem(p + 1 + g, N_DEV))
        for t in range(NTT):
            loads = [None, None]
            for ring in (0, 1):
                ag_recv_wait(ring, g, t)
                if g < N_DEV - 2:
                    ag_send(ring, g + 1, t, ag_tile(ring, g, t))
                cp = pltpu.make_async_copy(
                    ag_tile(ring, g, t), va16[ring], local_sems.at[2 * ring])
                cp.start()
                loads[ring] = cp
            stores = [None, None]
            for ring in (0, 1):
                loads[ring].wait()
                vf32[ring][...] = va16[ring][...].astype(jnp.float32) * 4.0
                st = pltpu.make_async_copy(
                    vf32[ring], out_tile(ring, rcv_chunk[ring], t),
                    local_sems.at[8 + ring])
                st.start()
                stores[ring] = st
            for ring in (0, 1):
                stores[ring].wait()

    for rdma in started:
        rdma.wait_send()


def kernel(x, w_mat, scale_x, scale_w):
    alpha4 = (scale_x[0] * scale_w[0] * 0.25).astype(jnp.float32)
    alpha4 = alpha4.reshape(1, 1)

    out = pl.pallas_call(
        _ar_body,
        out_shape=[
            jax.ShapeDtypeStruct((M, N), jnp.float32),
            jax.ShapeDtypeStruct((2, N_DEV - 1, CHUNK, N2), CDT),
            jax.ShapeDtypeStruct((2, N_DEV - 1, CHUNK, N2), CDT),
            jax.ShapeDtypeStruct((2, CHUNK, N2), CDT),
            jax.ShapeDtypeStruct((2, CHUNK, N2), CDT),
        ],
        in_specs=[
            pl.BlockSpec(memory_space=pl.ANY),
            pl.BlockSpec(memory_space=pltpu.VMEM),
            pl.BlockSpec(memory_space=pltpu.SMEM),
        ],
        out_specs=[pl.BlockSpec(memory_space=pl.ANY)] * 5,
        scratch_shapes=[
            pltpu.SemaphoreType.DMA((2, N_DEV - 1, NTT)),
            pltpu.SemaphoreType.DMA((2, N_DEV - 1, NTT)),
            pltpu.SemaphoreType.DMA((2, N_DEV - 1, NTT)),
            pltpu.SemaphoreType.DMA((2, N_DEV - 1, NTT)),
            pltpu.SemaphoreType.DMA((10,)),
            pltpu.VMEM((TILE, N2), CDT),
            pltpu.VMEM((TILE, N2), CDT),
            pltpu.VMEM((TILE, N2), CDT),
            pltpu.VMEM((TILE, N2), CDT),
            pltpu.VMEM((TILE, N2), CDT),
            pltpu.VMEM((TILE, N2), CDT),
            pltpu.VMEM((TILE, N2), jnp.float32),
            pltpu.VMEM((TILE, N2), jnp.float32),
            pltpu.VMEM((TILE, K), jnp.int8),
            pltpu.VMEM((TILE, K), jnp.int8),
        ],
        compiler_params=pltpu.CompilerParams(collective_id=0),
    )(x, w_mat, alpha4)[0]
    return out


# device time: 645849 ns/iter; 3.8236x vs baseline; 1.0045x over previous
import jax
import jax.numpy as jnp
from jax import lax
from jax.experimental import pallas as pl
from jax.experimental.pallas import tpu as pltpu

N_DEV = 4
M = 4096
K = 1024
N = 8192
N2 = N // 2
CHUNK = M // N_DEV
NTT = 4
TILE = CHUNK // NTT
CDT = jnp.bfloat16
NSLOT = 4
NFS = 2


def _ar_body(x_ref, w_ref, alpha_ref, out_ref, rs_comm, ag_comm,
             rs_send_sems, rs_recv_sems, ag_send_sems, ag_recv_sems,
             local_sems, vs_ref, vf_ref, va_ref, vb_ref, xi_ref):
    p = lax.axis_index("i")
    right = lax.rem(p + 1, N_DEV)
    left = lax.rem(p + N_DEV - 1, N_DEV)
    alpha4 = alpha_ref[0, 0]

    barrier = pltpu.get_barrier_semaphore()
    for nbr in (left, right):
        pl.semaphore_signal(barrier, inc=1, device_id=(nbr,),
                            device_id_type=pl.DeviceIdType.MESH)
    pl.semaphore_wait(barrier, 2)

    send_idx = [0, 0]
    pending_send = {}
    store_idx = [0, 0]
    pending_store = {}
    fw_sends = []

    def rs_tile(ring, h, t):
        return rs_comm.at[ring, h].at[pl.ds(t * TILE, TILE), :]

    def ag_tile(ring, g, t):
        return ag_comm.at[ring, g].at[pl.ds(t * TILE, TILE), :]

    def out_tile(ring, chunk, t):
        return out_ref.at[pl.ds(chunk * CHUNK + t * TILE, TILE),
                          pl.ds((0, N2)[ring], N2)]

    def fresh_slot(ring):
        s = send_idx[ring] % NSLOT
        send_idx[ring] += 1
        prev = pending_send.pop((ring, s), None)
        if prev is not None:
            prev.wait_send()
        return s

    def fresh_fslot(ring):
        fs = store_idx[ring] % NFS
        store_idx[ring] += 1
        prev = pending_store.pop((ring, fs), None)
        if prev is not None:
            prev.wait()
        return fs

    def rs_send(ring, h, t, slot):
        rdma = pltpu.make_async_remote_copy(
            src_ref=vs_ref.at[ring, slot], dst_ref=rs_tile(ring, h, t),
            send_sem=rs_send_sems.at[ring, h, t],
            recv_sem=rs_recv_sems.at[ring, h, t],
            device_id=((right, left)[ring],),
            device_id_type=pl.DeviceIdType.MESH,
        )
        rdma.start()
        pending_send[(ring, slot)] = rdma

    def rs_recv_wait(ring, h, t):
        pltpu.make_async_remote_copy(
            src_ref=rs_tile(ring, h, t), dst_ref=rs_tile(ring, h, t),
            send_sem=rs_send_sems.at[ring, h, t],
            recv_sem=rs_recv_sems.at[ring, h, t],
            device_id=((left, right)[ring],),
            device_id_type=pl.DeviceIdType.MESH,
        ).wait_recv()

    def ag_send(ring, g, t, src, slot=None):
        rdma = pltpu.make_async_remote_copy(
            src_ref=src, dst_ref=ag_tile(ring, g, t),
            send_sem=ag_send_sems.at[ring, g, t],
            recv_sem=ag_recv_sems.at[ring, g, t],
            device_id=((right, left)[ring],),
            device_id_type=pl.DeviceIdType.MESH,
        )
        rdma.start()
        if slot is None:
            fw_sends.append(rdma)
        else:
            pending_send[(ring, slot)] = rdma

    def ag_recv_wait(ring, g, t):
        pltpu.make_async_remote_copy(
            src_ref=ag_tile(ring, g, t), dst_ref=ag_tile(ring, g, t),
            send_sem=ag_send_sems.at[ring, g, t],
            recv_sem=ag_recv_sems.at[ring, g, t],
            device_id=((left, right)[ring],),
            device_id_type=pl.DeviceIdType.MESH,
        ).wait_recv()

    def partial_tile(ring, chunk, t):
        row0 = chunk * CHUNK + t * TILE
        cp = pltpu.make_async_copy(
            x_ref.at[pl.ds(row0, TILE), :], xi_ref.at[ring],
            local_sems.at[2 + ring])
        cp.start()
        cp.wait()
        acc = lax.dot_general(
            xi_ref[ring], w_ref[:, (0, N2)[ring]:(N2, N)[ring]],
            dimension_numbers=(((1,), (0,)), ((), ())),
            preferred_element_type=jnp.int32,
        )
        return (acc.astype(jnp.float32) * alpha4).astype(CDT)

    send_chunk = (lax.rem(p + N_DEV - 1, N_DEV), lax.rem(p + 1, N_DEV))
    for t in range(NTT):
        for ring in (0, 1):
            s = fresh_slot(ring)
            vs_ref[ring, s, :, :] = partial_tile(ring, send_chunk[ring], t)
            rs_send(ring, 0, t, s)

    for h in range(N_DEV - 1):
        rcv_chunk = (lax.rem(p + 2 * N_DEV - 2 - h, N_DEV),
                     lax.rem(p + 2 + h, N_DEV))
        last = h == N_DEV - 2
        for t in range(NTT):
            for ring in (0, 1):
                vb_ref[ring, :, :] = partial_tile(ring, rcv_chunk[ring], t)
            loads = [None, None]
            for ring in (0, 1):
                rs_recv_wait(ring, h, t)
                cp = pltpu.make_async_copy(
                    rs_tile(ring, h, t), va_ref.at[ring],
                    local_sems.at[ring])
                cp.start()
                loads[ring] = cp
            for ring in (0, 1):
                loads[ring].wait()
                acc = (va_ref[ring].astype(jnp.float32)
                       + vb_ref[ring].astype(jnp.float32))
                s = fresh_slot(ring)
                if not last:
                    vs_ref[ring, s, :, :] = acc.astype(CDT)
                    rs_send(ring, h + 1, t, s)
                else:
                    y = jnp.maximum(acc, 0.0)
                    vs_ref[ring, s, :, :] = y.astype(CDT)
                    ag_send(ring, 0, t, vs_ref.at[ring, s], slot=s)
                    fs = fresh_fslot(ring)
                    vf_ref[ring, fs, :, :] = y * 4.0
                    st = pltpu.make_async_copy(
                        vf_ref.at[ring, fs],
                        out_tile(ring, rcv_chunk[ring], t),
                        local_sems.at[4 + NFS * ring + fs])
                    st.start()
                    pending_store[(ring, fs)] = st

    for g in range(N_DEV - 1):
        rcv_chunk = (lax.rem(p + 2 * N_DEV - 1 - g, N_DEV),
                     lax.rem(p + 1 + g, N_DEV))
        for t in range(NTT):
            loads = [None, None]
            for ring in (0, 1):
                ag_recv_wait(ring, g, t)
                if g < N_DEV - 2:
                    ag_send(ring, g + 1, t, ag_tile(ring, g, t))
                cp = pltpu.make_async_copy(
                    ag_tile(ring, g, t), va_ref.at[ring],
                    local_sems.at[ring])
                cp.start()
                loads[ring] = cp
            for ring in (0, 1):
                loads[ring].wait()
                fs = fresh_fslot(ring)
                vf_ref[ring, fs, :, :] = va_ref[ring].astype(jnp.float32) * 4.0
                st = pltpu.make_async_copy(
                    vf_ref.at[ring, fs],
                    out_tile(ring, rcv_chunk[ring], t),
                    local_sems.at[4 + NFS * ring + fs])
                st.start()
                pending_store[(ring, fs)] = st

    for rdma in pending_send.values():
        rdma.wait_send()
    for rdma in fw_sends:
        rdma.wait_send()
    for st in pending_store.values():
        st.wait()


def kernel(x, w_mat, scale_x, scale_w):
    alpha4 = (scale_x[0] * scale_w[0] * 0.25).astype(jnp.float32)
    alpha4 = alpha4.reshape(1, 1)

    out = pl.pallas_call(
        _ar_body,
        out_shape=[
            jax.ShapeDtypeStruct((M, N), jnp.float32),
            jax.ShapeDtypeStruct((2, N_DEV - 1, CHUNK, N2), CDT),
            jax.ShapeDtypeStruct((2, N_DEV - 1, CHUNK, N2), CDT),
        ],
        in_specs=[
            pl.BlockSpec(memory_space=pl.ANY),
            pl.BlockSpec(memory_space=pltpu.VMEM),
            pl.BlockSpec(memory_space=pltpu.SMEM),
        ],
        out_specs=[pl.BlockSpec(memory_space=pl.ANY)] * 3,
        scratch_shapes=[
            pltpu.SemaphoreType.DMA((2, N_DEV - 1, NTT)),
            pltpu.SemaphoreType.DMA((2, N_DEV - 1, NTT)),
            pltpu.SemaphoreType.DMA((2, N_DEV - 1, NTT)),
            pltpu.SemaphoreType.DMA((2, N_DEV - 1, NTT)),
            pltpu.SemaphoreType.DMA((4 + 2 * NFS,)),
            pltpu.VMEM((2, NSLOT, TILE, N2), CDT),
            pltpu.VMEM((2, NFS, TILE, N2), jnp.float32),
            pltpu.VMEM((2, TILE, N2), CDT),
            pltpu.VMEM((2, TILE, N2), CDT),
            pltpu.VMEM((2, TILE, K), jnp.int8),
        ],
        compiler_params=pltpu.CompilerParams(
            collective_id=0,
            vmem_limit_bytes=100 * 1024 * 1024,
        ),
    )(x, w_mat, alpha4)[0]
    return out
